# baseline (speedup 1.0000x reference)
import functools

import jax
import jax.numpy as jnp
from jax import lax
from jax.experimental import pallas as pl
from jax.experimental.pallas import tpu as pltpu
from jax.experimental.pallas import tpu_sc as plsc

_B, _T, _D = 1024, 50, 300
_CHUNK = 2


def _sc_copy(v_hbm, o_hbm, buf0, buf1, sem_in, sem_out, nc):
    wid = lax.axis_index("s") * nc + lax.axis_index("c")
    rows_per_w = _B // (nc * 16)
    n = rows_per_w // _CHUNK
    base = wid * rows_per_w

    def in_copy(i, buf, slot):
        return pltpu.make_async_copy(
            v_hbm.at[pl.ds(base + i * _CHUNK, _CHUNK)], buf, sem_in.at[slot]
        )

    def out_copy(i, buf, slot):
        return pltpu.make_async_copy(
            buf, o_hbm.at[pl.ds(base + i * _CHUNK, _CHUNK)], sem_out.at[slot]
        )

    in_copy(0, buf0, 0).start()

    def pair(i):
        ii = i * 2
        in_copy(ii, buf0, 0).wait()
        out_copy(ii, buf0, 0).start()

        @pl.when(ii + 1 < n)
        def _():
            in_copy(ii + 1, buf1, 1).start()
            in_copy(ii + 1, buf1, 1).wait()

        @pl.when(ii + 2 < n)
        def _():
            in_copy(ii + 2, buf0, 0).start()

        @pl.when(ii + 1 < n)
        def _():
            out_copy(ii + 1, buf1, 1).start()
        out_copy(ii, buf0, 0).wait()

        @pl.when(ii + 1 < n)
        def _():
            out_copy(ii + 1, buf1, 1).wait()

    pl.loop(0, n // 2)(pair)


def kernel(video, ques, attr, emb):
    del ques, attr, emb
    info = plsc.get_sparse_core_info()
    nc = info.num_cores
    mesh = plsc.VectorSubcoreMesh(core_axis_name="c", subcore_axis_name="s")
    k = pl.kernel(
        functools.partial(_sc_copy, nc=nc),
        out_type=jax.ShapeDtypeStruct((_B, _T, _D), jnp.float32),
        mesh=mesh,
        scratch_types=[
            pltpu.VMEM((_CHUNK, _T, _D), jnp.float32),
            pltpu.VMEM((_CHUNK, _T, _D), jnp.float32),
            pltpu.SemaphoreType.DMA((2,)),
            pltpu.SemaphoreType.DMA((2,)),
        ],
        compiler_params=pltpu.CompilerParams(skip_device_barrier=True),
    )
    return k(video)
